# Initial kernel scaffold; baseline (speedup 1.0000x reference)
import jax
import jax.numpy as jnp
from jax.experimental import pallas as pl


def kernel(x, vault_keys):
    raise NotImplementedError("write your pallas kernel here")


# speedup vs baseline: 1.4854x; 1.4854x over previous
import functools
import math

import jax
import jax.numpy as jnp
from jax.experimental import pallas as pl
from jax.experimental.pallas import tpu as pltpu

_BLOCK = 8192
_TOPK = 8
_NEG_INF = float("-inf")
_IMAX = jnp.iinfo(jnp.int32).max


def _scan_body(x_ref, vk_ref, topv_ref, topi_ref, *, n_keys, block):
    b = pl.program_id(0)
    nb = pl.num_programs(0)
    d = x_ref.shape[1]
    q = x_ref.shape[0]

    ri = jax.lax.broadcasted_iota(jnp.int32, (d, d), 0) // 4
    ci = jax.lax.broadcasted_iota(jnp.int32, (d, d), 1) // 4
    g_mask = (ri == ci).astype(jnp.float32)

    x = x_ref[...]
    xn = jnp.sqrt(
        jax.lax.dot_general(x * x, g_mask, (((1,), (0,)), ((), ())),
                            preferred_element_type=jnp.float32,
                            precision=jax.lax.Precision.HIGHEST) + 1e-8)
    qn = (x / xn) * 0.5

    kb = vk_ref[...]
    kn = jnp.sqrt(
        jax.lax.dot_general(kb * kb, g_mask, (((1,), (0,)), ((), ())),
                            preferred_element_type=jnp.float32,
                            precision=jax.lax.Precision.HIGHEST) + 1e-8)
    knb = kb / kn

    scores = jax.lax.dot_general(qn, knb, (((1,), (1,)), ((), ())),
                                 preferred_element_type=jnp.float32)

    base = b * block
    gidx = jax.lax.broadcasted_iota(jnp.int32, (q, block), 1) + base
    valid = gidx < n_keys
    scores = jnp.where(valid, scores, _NEG_INF)
    gidx = jnp.where(valid, gidx, _IMAX)

    @pl.when(b == 0)
    def _init():
        topv_ref[...] = jnp.full((q, _TOPK), _NEG_INF, jnp.float32)
        topi_ref[...] = jnp.full((q, _TOPK), _IMAX, jnp.int32)

    cs = jnp.concatenate([scores, topv_ref[...]], axis=1)
    ci_all = jnp.concatenate([gidx, topi_ref[...]], axis=1)

    lane = jax.lax.broadcasted_iota(jnp.int32, (q, _TOPK), 1)
    rv = jnp.full((q, _TOPK), _NEG_INF, jnp.float32)
    ri_out = jnp.full((q, _TOPK), _IMAX, jnp.int32)
    for j in range(_TOPK):
        m = jnp.max(cs, axis=1)
        sel = cs == m[:, None]
        cand = jnp.where(sel, ci_all, _IMAX)
        pick = jnp.min(cand, axis=1)
        chosen = sel & (ci_all == pick[:, None])
        rv = jnp.where(lane == j, m[:, None], rv)
        ri_out = jnp.where(lane == j, pick[:, None], ri_out)
        cs = jnp.where(chosen, _NEG_INF, cs)

    topv_ref[...] = rv
    topi_ref[...] = ri_out


def _readout_body(idx_ref, tv_ref, *row_and_out_refs):
    row_refs = row_and_out_refs[:_TOPK]
    out_ref = row_and_out_refs[_TOPK]
    tv = tv_ref[...]
    m = jnp.max(tv)
    e = jnp.exp((tv - m) * 10.0)
    w = e / jnp.sum(e)
    acc = jnp.zeros(out_ref.shape, jnp.float32)
    for j in range(_TOPK):
        acc = acc + w[:, :, j:j + 1] * row_refs[j][...]
    out_ref[...] = acc


def kernel(x, vault_keys):
    n_keys, d = vault_keys.shape
    q = x.shape[0]
    block = _BLOCK
    nb = math.ceil(n_keys / block)

    topv, topi = pl.pallas_call(
        functools.partial(_scan_body, n_keys=n_keys, block=block),
        grid=(nb,),
        in_specs=[
            pl.BlockSpec((q, d), lambda b: (0, 0)),
            pl.BlockSpec((block, d), lambda b: (b, 0)),
        ],
        out_specs=[
            pl.BlockSpec((q, _TOPK), lambda b: (0, 0)),
            pl.BlockSpec((q, _TOPK), lambda b: (0, 0)),
        ],
        out_shape=[
            jax.ShapeDtypeStruct((q, _TOPK), jnp.float32),
            jax.ShapeDtypeStruct((q, _TOPK), jnp.int32),
        ],
    )(x, vault_keys)

    v3 = vault_keys.reshape(n_keys, 1, d)
    tv3 = topv.reshape(q, 1, _TOPK)
    idx_flat = topi.reshape(-1)

    def _row_spec(j):
        return pl.BlockSpec((1, 1, d),
                            lambda qi, idx, j=j: (idx[qi * _TOPK + j], 0, 0))

    out3 = pl.pallas_call(
        _readout_body,
        grid_spec=pltpu.PrefetchScalarGridSpec(
            num_scalar_prefetch=1,
            grid=(q,),
            in_specs=[pl.BlockSpec((1, 1, _TOPK), lambda qi, idx: (qi, 0, 0))]
            + [_row_spec(j) for j in range(_TOPK)],
            out_specs=pl.BlockSpec((1, 1, d), lambda qi, idx: (qi, 0, 0)),
        ),
        out_shape=jax.ShapeDtypeStruct((q, 1, d), jnp.float32),
    )(idx_flat, tv3, *([v3] * _TOPK))

    return out3.reshape(q, d)
